# Initial kernel scaffold; baseline (speedup 1.0000x reference)
import jax
import jax.numpy as jnp
from jax.experimental import pallas as pl


def kernel(logits, targets, batchs):
    raise NotImplementedError("write your pallas kernel here")


# speedup vs baseline: 8.9523x; 8.9523x over previous
import functools

import jax
import jax.numpy as jnp
from jax import lax
from jax.experimental import pallas as pl
from jax.experimental.pallas import tpu as pltpu

_N = 16384
_K = max(1, int(0.7 * _N))


def _tc_body(x_ref, t0_ref, t1_ref, out_ref):
    x = x_ref[...]
    t = (t1_ref[...] > t0_ref[...]).astype(jnp.float32)
    ax = jnp.abs(x)
    losses = jnp.maximum(x, 0.0) - x * t + jnp.log1p(jnp.exp(-ax))
    bits = lax.bitcast_convert_type(losses, jnp.int32)

    def bs_step(_, lohi):
        lo, hi = lohi
        mid = lo + (hi - lo) // 2
        cnt = jnp.sum((bits <= mid).astype(jnp.int32))
        return jnp.where(cnt >= _K, lo, mid + 1), jnp.where(cnt >= _K, mid, hi)

    lo, hi = lax.fori_loop(0, 31, bs_step, (jnp.int32(0), jnp.int32(0x7F800000)))
    thr_bits = lo
    below = bits < thr_bits
    cb = jnp.sum(below.astype(jnp.int32))
    sb = jnp.sum(jnp.where(below, losses, 0.0))
    thr = lax.bitcast_convert_type(thr_bits, jnp.float32)
    total = sb + thr * (_K - cb).astype(jnp.float32)
    out_ref[0, 0] = total / jnp.float32(_K)


def kernel(logits, targets, batchs):
    x = logits.reshape(128, 128)
    t0 = targets[:, 0].reshape(128, 128)
    t1 = targets[:, 1].reshape(128, 128)
    out = pl.pallas_call(
        _tc_body,
        out_shape=jax.ShapeDtypeStruct((1, 1), jnp.float32),
        out_specs=pl.BlockSpec(memory_space=pltpu.SMEM),
    )(x, t0, t1)
    return out[0, 0]
